# baseline (speedup 1.0000x reference)
import functools
import math

import jax
import jax.numpy as jnp
from jax import lax
from jax.experimental import pallas as pl
from jax.experimental.pallas import tpu as pltpu
from jax.experimental.pallas import tpu_sc as plsc

_D = 128
_B = 1024
_N = 100000
_TEMP = 0.05
_INV_TEMP = 1.0 / _TEMP
_SCALE = _INV_TEMP / math.log(2.0)
_BLK = 10000
_STEPS = _N // _BLK
_CE = 640
_SCH_C = (127.0 - 0.04068408365780063) * 8388608.0


def _target_dots(targets, inputs, features):
    info = plsc.get_sparse_core_info()
    nc, ns = info.num_cores, info.num_subcores
    nw = nc * ns
    bpw = _B // nw
    mesh = plsc.VectorSubcoreMesh(core_axis_name="c", subcore_axis_name="s")

    @functools.partial(
        pl.kernel,
        out_type=jax.ShapeDtypeStruct((_B,), jnp.float32),
        mesh=mesh,
        scratch_types=[
            pltpu.VMEM((bpw,), jnp.int32),
            pltpu.VMEM((bpw, _D), jnp.float32),
            pltpu.VMEM((bpw, _D), jnp.float32),
            pltpu.VMEM((bpw,), jnp.float32),
            pltpu.SemaphoreType.DMA,
        ],
    )
    def dot_k(tgt_hbm, x_hbm, feat_hbm, td_hbm, idx_v, rows_v, x_v, td_v, sem):
        lane = lax.iota(jnp.int32, 16)
        wid = lax.axis_index("s") * nc + lax.axis_index("c")
        base = wid * bpw
        pltpu.sync_copy(tgt_hbm.at[pl.ds(base, bpw)], idx_v)
        cp = pltpu.async_copy(feat_hbm.at[idx_v], rows_v, sem)
        pltpu.sync_copy(x_hbm.at[pl.ds(base, bpw)], x_v)
        cp.wait()
        def row_dot(r):
            acc = jnp.zeros((16,), jnp.float32)
            for c in range(_D // 16):
                xv = x_v[r, pl.ds(c * 16, 16)]
                acc = acc + xv * rows_v[r, pl.ds(c * 16, 16)]
            for sh in (8, 4, 2, 1):
                acc = acc + lax.gather(
                    acc,
                    jnp.bitwise_xor(lane, sh)[:, None],
                    lax.GatherDimensionNumbers(
                        offset_dims=(),
                        collapsed_slice_dims=(0,),
                        start_index_map=(0,)),
                    (1,),
                    mode=lax.GatherScatterMode.PROMISE_IN_BOUNDS)
            return acc

        def grp_body(grp, carry):
            def k_body(k, res):
                return jnp.where(lane == k, row_dot(grp * 16 + k), res)
            res = lax.fori_loop(0, 16, k_body, jnp.zeros((16,), jnp.float32))
            td_v[pl.ds(grp * 16, 16)] = res
            return carry

        lax.fori_loop(0, bpw // 16, grp_body, jnp.int32(0))
        pltpu.sync_copy(td_v, td_hbm.at[pl.ds(base, bpw)])

    return dot_k(targets, inputs, features)


def _tc_body(x_ref, f_ref, out_ref, invn_ref, xn_ref, s_ref):
    step = pl.program_id(0)

    @pl.when(step == 0)
    def _init():
        x = x_ref[...]
        n2 = jnp.sum(x * x, axis=1, keepdims=True)
        inv = jnp.minimum(lax.rsqrt(jnp.maximum(n2, 1e-30)), 1e12)
        invn_ref[...] = inv
        xn_ref[...] = (x * (inv * _SCALE)).astype(jnp.float8_e4m3fn)
        s_ref[...] = jnp.zeros_like(s_ref)

    fb = f_ref[...].astype(jnp.float8_e4m3fn)
    part = lax.dot_general(
        fb, xn_ref[...], (((1,), (1,)), ((), ())),
        preferred_element_type=jnp.float32,
    )
    ec = jnp.exp2(part[:, :_CE])
    vc = lax.bitcast_convert_type(
        (part[:, _CE:] * 8388608.0 + _SCH_C).astype(jnp.int32),
        jnp.float32)
    s_ref[:, :_CE] += jnp.sum(ec.reshape(_BLK // 8, 8, _CE), axis=0)
    s_ref[:, _CE:] += jnp.sum(vc.reshape(_BLK // 8, 8, _B - _CE), axis=0)

    @pl.when(step == _STEPS - 1)
    def _fini():
        s_row = jnp.sum(s_ref[...], axis=0, keepdims=True)
        out_ref[0, 0] = jnp.sum(jnp.log(s_row))


def kernel(inputs, targets, features):
    lse_sum, invn = pl.pallas_call(
        _tc_body,
        grid=(_STEPS,),
        in_specs=[
            pl.BlockSpec((_B, _D), lambda i: (0, 0)),
            pl.BlockSpec((_BLK, _D), lambda i: (i, 0)),
        ],
        out_specs=[
            pl.BlockSpec(memory_space=pltpu.SMEM),
            pl.BlockSpec((_B, 1), lambda i: (0, 0)),
        ],
        out_shape=[
            jax.ShapeDtypeStruct((1, 1), jnp.float32),
            jax.ShapeDtypeStruct((_B, 1), jnp.float32),
        ],
        scratch_shapes=[
            pltpu.VMEM((_B, _D), jnp.float8_e4m3fn),
            pltpu.VMEM((8, _B), jnp.float32),
        ],
        compiler_params=pltpu.CompilerParams(
            dimension_semantics=("arbitrary",),
        ),
    )(inputs, features)
    t_raw = _target_dots(targets, inputs, features)
    tgt_sum = jnp.sum(t_raw * invn[:, 0]) * _INV_TEMP
    return (lse_sum[0, 0] - tgt_sum) * (1.0 / _B)


# speedup vs baseline: 1.0643x; 1.0643x over previous
import functools
import math

import jax
import jax.numpy as jnp
from jax import lax
from jax.experimental import pallas as pl
from jax.experimental.pallas import tpu as pltpu
from jax.experimental.pallas import tpu_sc as plsc

_D = 128
_B = 1024
_N = 100000
_TEMP = 0.05
_INV_TEMP = 1.0 / _TEMP
_SCALE = _INV_TEMP / math.log(2.0)
_BLK = 10000
_STEPS = _N // _BLK
_CE = 768
_SCH_C = (127.0 - 0.04068408365780063) * 8388608.0


def _target_dots(targets, inputs, features):
    info = plsc.get_sparse_core_info()
    nc, ns = info.num_cores, info.num_subcores
    nw = nc * ns
    bpw = _B // nw
    mesh = plsc.VectorSubcoreMesh(core_axis_name="c", subcore_axis_name="s")

    @functools.partial(
        pl.kernel,
        out_type=jax.ShapeDtypeStruct((_B,), jnp.float32),
        mesh=mesh,
        scratch_types=[
            pltpu.VMEM((bpw,), jnp.int32),
            pltpu.VMEM((bpw, _D), jnp.float32),
            pltpu.VMEM((bpw, _D), jnp.float32),
            pltpu.VMEM((bpw,), jnp.float32),
            pltpu.SemaphoreType.DMA,
        ],
    )
    def dot_k(tgt_hbm, x_hbm, feat_hbm, td_hbm, idx_v, rows_v, x_v, td_v, sem):
        lane = lax.iota(jnp.int32, 16)
        wid = lax.axis_index("s") * nc + lax.axis_index("c")
        base = wid * bpw
        pltpu.sync_copy(tgt_hbm.at[pl.ds(base, bpw)], idx_v)
        cp = pltpu.async_copy(feat_hbm.at[idx_v], rows_v, sem)
        pltpu.sync_copy(x_hbm.at[pl.ds(base, bpw)], x_v)
        cp.wait()
        def row_dot(r):
            acc = jnp.zeros((16,), jnp.float32)
            for c in range(_D // 16):
                xv = x_v[r, pl.ds(c * 16, 16)]
                acc = acc + xv * rows_v[r, pl.ds(c * 16, 16)]
            for sh in (8, 4, 2, 1):
                acc = acc + lax.gather(
                    acc,
                    jnp.bitwise_xor(lane, sh)[:, None],
                    lax.GatherDimensionNumbers(
                        offset_dims=(),
                        collapsed_slice_dims=(0,),
                        start_index_map=(0,)),
                    (1,),
                    mode=lax.GatherScatterMode.PROMISE_IN_BOUNDS)
            return acc

        def grp_body(grp, carry):
            def k_body(k, res):
                return jnp.where(lane == k, row_dot(grp * 16 + k), res)
            res = lax.fori_loop(0, 16, k_body, jnp.zeros((16,), jnp.float32))
            td_v[pl.ds(grp * 16, 16)] = res
            return carry

        lax.fori_loop(0, bpw // 16, grp_body, jnp.int32(0))
        pltpu.sync_copy(td_v, td_hbm.at[pl.ds(base, bpw)])

    return dot_k(targets, inputs, features)


def _tc_body(x_ref, f_ref, out_ref, invn_ref, xn_ref, s_ref):
    step = pl.program_id(0)

    @pl.when(step == 0)
    def _init():
        x = x_ref[...]
        n2 = jnp.sum(x * x, axis=1, keepdims=True)
        inv = jnp.minimum(lax.rsqrt(jnp.maximum(n2, 1e-30)), 1e12)
        invn_ref[...] = inv
        xn_ref[...] = (x * (inv * _SCALE)).astype(jnp.float8_e4m3fn)
        s_ref[...] = jnp.zeros_like(s_ref)

    fb = f_ref[...].astype(jnp.float8_e4m3fn)
    part = lax.dot_general(
        fb, xn_ref[...], (((1,), (1,)), ((), ())),
        preferred_element_type=jnp.float32,
    )
    ec = jnp.exp2(part[:, :_CE])
    vc = lax.bitcast_convert_type(
        (part[:, _CE:] * 8388608.0 + _SCH_C).astype(jnp.int32),
        jnp.float32)
    s_ref[:, :_CE] += jnp.sum(ec.reshape(_BLK // 8, 8, _CE), axis=0)
    s_ref[:, _CE:] += jnp.sum(vc.reshape(_BLK // 8, 8, _B - _CE), axis=0)

    @pl.when(step == _STEPS - 1)
    def _fini():
        s_row = jnp.sum(s_ref[...], axis=0, keepdims=True)
        out_ref[0, 0] = jnp.sum(jnp.log(s_row))


def kernel(inputs, targets, features):
    lse_sum, invn = pl.pallas_call(
        _tc_body,
        grid=(_STEPS,),
        in_specs=[
            pl.BlockSpec((_B, _D), lambda i: (0, 0)),
            pl.BlockSpec((_BLK, _D), lambda i: (i, 0)),
        ],
        out_specs=[
            pl.BlockSpec(memory_space=pltpu.SMEM),
            pl.BlockSpec((_B, 1), lambda i: (0, 0)),
        ],
        out_shape=[
            jax.ShapeDtypeStruct((1, 1), jnp.float32),
            jax.ShapeDtypeStruct((_B, 1), jnp.float32),
        ],
        scratch_shapes=[
            pltpu.VMEM((_B, _D), jnp.float8_e4m3fn),
            pltpu.VMEM((8, _B), jnp.float32),
        ],
        compiler_params=pltpu.CompilerParams(
            dimension_semantics=("arbitrary",),
        ),
    )(inputs, features)
    t_raw = _target_dots(targets, inputs, features)
    tgt_sum = jnp.sum(t_raw * invn[:, 0]) * _INV_TEMP
    return (lse_sum[0, 0] - tgt_sum) * (1.0 / _B)
